# baseline (speedup 1.0000x reference)
import jax
import jax.numpy as jnp
from jax import lax
from jax.experimental import pallas as pl
from jax.experimental.pallas import tpu as pltpu

_N = 32768
_B = 16
_BN = 8192
_NB = _N // _BN
_NEG = -1.0e30


def _dotg(a, b, dims):
  return lax.dot_general(a, b, (dims, ((), ())),
                         preferred_element_type=jnp.float32)


def _body(x_ref, ids_ref, idsf_ref, sl_ref, p_ref,
          a_ref, wxt_ref, ct_ref, w2c_ref, m_ref, s_ref, e_ref):
  i = pl.program_id(0)

  @pl.when(i == 0)
  def _init():
    wu1 = p_ref[0:500, :]
    wu2 = p_ref[504:597, :]
    w1_top = p_ref[600:750, :]
    w1_bot = p_ref[750:900, :]
    bias = p_ref[904:912, :]
    bu1 = bias[0:1, :]
    bu2 = bias[1:2, :]
    bw1 = bias[2:3, :]
    w2r = bias[3:4, :]
    ones11 = jnp.ones((1, 1), jnp.float32)
    wxt_ref[...] = _dotg(w1_bot, wu2, ((0,), (1,)))
    u1 = jnp.dot(sl_ref[...], wu1,
                 preferred_element_type=jnp.float32) + bu1
    ct_ref[...] = (_dotg(w1_top, u1, ((0,), (1,)))
                   + _dotg(w1_bot, bu2, ((0,), (1,)))
                   + _dotg(bw1, ones11, ((0,), (0,))))
    w2c_ref[...] = _dotg(w2r, ones11, ((0,), (0,)))
    m_ref[...] = jnp.full((_B, 1), _NEG, jnp.float32)
    s_ref[...] = jnp.zeros((_B, 1), jnp.float32)

  @pl.when(i < _NB)
  def _scores():
    ids = ids_ref[...]
    ohb = ids == lax.broadcasted_iota(jnp.int32, (_B, 1), 0)
    pre = _dotg(wxt_ref[...], x_ref[...], ((1,), (1,)))
    cg = _dotg(ct_ref[...], ohb.astype(jnp.float32), ((1,), (0,)))
    ht = jnp.tanh(pre + cg)
    et = (jnp.sum(ht * w2c_ref[...], axis=0, keepdims=True)
          + p_ref[908:909, 0:1])
    e_ref[pl.ds(i, 1), :] = et
    m_part = jnp.max(jnp.where(ohb, et, _NEG), axis=1, keepdims=True)
    s_part = jnp.sum(jnp.where(ohb, jnp.exp(et), 0.0), axis=1, keepdims=True)
    m_ref[...] = jnp.maximum(m_ref[...], m_part)
    s_ref[...] = s_ref[...] + s_part

  @pl.when(i == _NB)
  def _normalize():
    m = jnp.maximum(m_ref[...], -80.0)
    em = jnp.exp(-m)
    s = s_ref[...]
    q = jnp.where(s > 0.0, em / (em * s + 1e-16), 0.0)
    iota_b = lax.broadcasted_iota(jnp.int32, (_B, 1), 0)
    for j in range(_NB):
      ids_j = idsf_ref[0:1, j * _BN:(j + 1) * _BN]
      ohf = (ids_j == iota_b).astype(jnp.float32)
      qg = _dotg(q, ohf, ((0,), (0,)))
      a_ref[0:1, j * _BN:(j + 1) * _BN] = (
          jnp.exp(e_ref[j:j + 1, :]) * qg)


def kernel(x_t, x_t_batch, smile_latent, W_U1, b_U1, W_U2, b_U2,
           W_W1, b_W1, W_W2, b_W2):
  ids = x_t_batch.astype(jnp.int32).reshape(1, _N)
  f32 = jnp.float32
  params = jnp.concatenate([
      W_U1,
      jnp.zeros((4, 150), f32),
      W_U2,
      jnp.zeros((3, 150), f32),
      W_W1,
      jnp.zeros((4, 150), f32),
      b_U1.reshape(1, 150),
      b_U2.reshape(1, 150),
      b_W1.reshape(1, 150),
      W_W2.reshape(1, 150),
      jnp.zeros((1, 150), f32) + b_W2.reshape(1, 1),
      jnp.zeros((3, 150), f32),
  ], axis=0)
  last = _NB - 1
  alpha = pl.pallas_call(
      _body,
      grid=(_NB + 1,),
      in_specs=[
          pl.BlockSpec((_BN, 93), lambda i: (jnp.minimum(i, last), 0)),
          pl.BlockSpec((1, _BN), lambda i: (0, jnp.minimum(i, last))),
          pl.BlockSpec((1, _N), lambda i: (0, 0)),
          pl.BlockSpec((16, 500), lambda i: (0, 0)),
          pl.BlockSpec((912, 150), lambda i: (0, 0)),
      ],
      out_specs=pl.BlockSpec((1, _N), lambda i: (0, 0)),
      out_shape=jax.ShapeDtypeStruct((1, _N), jnp.float32),
      scratch_shapes=[
          pltpu.VMEM((150, 93), jnp.float32),
          pltpu.VMEM((150, _B), jnp.float32),
          pltpu.VMEM((150, 1), jnp.float32),
          pltpu.VMEM((_B, 1), jnp.float32),
          pltpu.VMEM((_B, 1), jnp.float32),
          pltpu.VMEM((_NB, _BN), jnp.float32),
      ],
  )(x_t, ids, ids, smile_latent, params)
  return alpha.reshape(_N, 1)


# speedup vs baseline: 1.0523x; 1.0523x over previous
import jax
import jax.numpy as jnp
from jax import lax
from jax.experimental import pallas as pl
from jax.experimental.pallas import tpu as pltpu

_N = 32768
_B = 16
_BN = 8192
_NB = _N // _BN
_NEG = -1.0e30


def _dotg(a, b, dims):
  return lax.dot_general(a, b, (dims, ((), ())),
                         preferred_element_type=jnp.float32)


def _body(x_hbm, ids_ref, idsf_ref, sl_ref, wu1_ref, bu1_ref, wu2_ref,
          bu2_ref, ww1_ref, bw1_ref, ww2_ref, bw2_ref,
          a_ref, wxt_ref, ct_ref, m_ref, s_ref, e_ref, xb0, xb1, s0, s1):
  i = pl.program_id(0)
  par = lax.rem(i, 2)

  def _copy(blk, buf, sem):
    return pltpu.make_async_copy(x_hbm.at[pl.ds(blk * _BN, _BN), :],
                                 buf, sem)

  @pl.when(i == 0)
  def _init():
    _copy(0, xb0, s0).start()
    ww1 = ww1_ref[...]
    w1_top = ww1[:150, :]
    w1_bot = ww1[150:, :]
    wxt_ref[...] = _dotg(w1_bot, wu2_ref[...], ((0,), (1,)))
    u1 = jnp.dot(sl_ref[...], wu1_ref[...],
                 preferred_element_type=jnp.float32) + bu1_ref[...]
    ct_ref[...] = (_dotg(w1_top, u1, ((0,), (1,)))
                   + _dotg(w1_bot, bu2_ref[...], ((0,), (1,)))
                   + bw1_ref[...])
    m_ref[...] = jnp.full((_B, 1), _NEG, jnp.float32)
    s_ref[...] = jnp.zeros((_B, 1), jnp.float32)

  nxt = i + 1

  @pl.when(jnp.logical_and(nxt < _NB, lax.rem(nxt, 2) == 1))
  def _prefetch_odd():
    _copy(nxt, xb1, s1).start()

  @pl.when(jnp.logical_and(nxt < _NB, lax.rem(nxt, 2) == 0))
  def _prefetch_even():
    _copy(nxt, xb0, s0).start()

  def _scores(xb, sem):
    _copy(i, xb, sem).wait()
    ids = ids_ref[...]
    ohb = ids == lax.broadcasted_iota(jnp.int32, (_B, 1), 0)
    pre = _dotg(wxt_ref[...], xb[...], ((1,), (1,)))
    cg = _dotg(ct_ref[...], ohb.astype(jnp.float32), ((1,), (0,)))
    ht = jnp.tanh(pre + cg)
    et = jnp.sum(ht * ww2_ref[...], axis=0, keepdims=True) + bw2_ref[...]
    e_ref[pl.ds(i, 1), :] = et
    m_part = jnp.max(jnp.where(ohb, et, _NEG), axis=1, keepdims=True)
    s_part = jnp.sum(jnp.where(ohb, jnp.exp(et), 0.0), axis=1, keepdims=True)
    m_ref[...] = jnp.maximum(m_ref[...], m_part)
    s_ref[...] = s_ref[...] + s_part

  @pl.when(jnp.logical_and(i < _NB, par == 0))
  def _scores_even():
    _scores(xb0, s0)

  @pl.when(jnp.logical_and(i < _NB, par == 1))
  def _scores_odd():
    _scores(xb1, s1)

  @pl.when(i == _NB)
  def _normalize():
    m = jnp.maximum(m_ref[...], -80.0)
    em = jnp.exp(-m)
    s = s_ref[...]
    q = jnp.where(s > 0.0, em / (em * s + 1e-16), 0.0)
    iota_b = lax.broadcasted_iota(jnp.int32, (_B, 1), 0)
    for j in range(_NB):
      ids_j = idsf_ref[0:1, j * _BN:(j + 1) * _BN]
      ohf = (ids_j == iota_b).astype(jnp.float32)
      qg = _dotg(q, ohf, ((0,), (0,)))
      a_ref[0:1, j * _BN:(j + 1) * _BN] = (
          jnp.exp(e_ref[j:j + 1, :]) * qg)


def kernel(x_t, x_t_batch, smile_latent, W_U1, b_U1, W_U2, b_U2,
           W_W1, b_W1, W_W2, b_W2):
  ids = x_t_batch.astype(jnp.int32).reshape(1, _N)
  last = _NB - 1
  alpha = pl.pallas_call(
      _body,
      grid=(_NB + 1,),
      in_specs=[
          pl.BlockSpec(memory_space=pltpu.MemorySpace.HBM),
          pl.BlockSpec((1, _BN), lambda i: (0, jnp.minimum(i, last))),
          pl.BlockSpec((1, _N), lambda i: (0, 0)),
          pl.BlockSpec((16, 500), lambda i: (0, 0)),
          pl.BlockSpec((500, 150), lambda i: (0, 0)),
          pl.BlockSpec((1, 150), lambda i: (0, 0)),
          pl.BlockSpec((93, 150), lambda i: (0, 0)),
          pl.BlockSpec((1, 150), lambda i: (0, 0)),
          pl.BlockSpec((300, 150), lambda i: (0, 0)),
          pl.BlockSpec((150, 1), lambda i: (0, 0)),
          pl.BlockSpec((150, 1), lambda i: (0, 0)),
          pl.BlockSpec((1, 1), lambda i: (0, 0)),
      ],
      out_specs=pl.BlockSpec((1, _N), lambda i: (0, 0)),
      out_shape=jax.ShapeDtypeStruct((1, _N), jnp.float32),
      scratch_shapes=[
          pltpu.VMEM((150, 93), jnp.float32),
          pltpu.VMEM((150, _B), jnp.float32),
          pltpu.VMEM((_B, 1), jnp.float32),
          pltpu.VMEM((_B, 1), jnp.float32),
          pltpu.VMEM((_NB, _BN), jnp.float32),
          pltpu.VMEM((_BN, 93), jnp.float32),
          pltpu.VMEM((_BN, 93), jnp.float32),
          pltpu.SemaphoreType.DMA,
          pltpu.SemaphoreType.DMA,
      ],
  )(x_t, ids, ids, smile_latent, W_U1, b_U1.reshape(1, 150), W_U2,
    b_U2.reshape(1, 150), W_W1, b_W1.reshape(150, 1), W_W2,
    b_W2.reshape(1, 1))
  return alpha.reshape(_N, 1)


# speedup vs baseline: 1.0695x; 1.0163x over previous
import jax
import jax.numpy as jnp
from jax import lax
from jax.experimental import pallas as pl
from jax.experimental.pallas import tpu as pltpu

_N = 32768
_B = 16
_BN = 8192
_NB = _N // _BN
_NEG = -1.0e30


def _dotg(a, b, dims):
  return lax.dot_general(a, b, (dims, ((), ())),
                         preferred_element_type=jnp.float32)


def _body(x_ref, ids_ref, idsf_ref, sl_ref, wu1_ref, bu1_ref, wu2_ref,
          bu2_ref, ww1_ref, bw1_ref, ww2_ref, bw2_ref,
          a_ref, wxt_ref, ct_ref, m_ref, s_ref, e_ref):
  i = pl.program_id(0)

  @pl.when(i == 0)
  def _init():
    ww1 = ww1_ref[...]
    w1_top = ww1[:150, :]
    w1_bot = ww1[150:, :]
    wxt_ref[...] = _dotg(w1_bot, wu2_ref[...], ((0,), (1,)))
    u1 = jnp.dot(sl_ref[...], wu1_ref[...],
                 preferred_element_type=jnp.float32) + bu1_ref[...]
    ct_ref[...] = (_dotg(w1_top, u1, ((0,), (1,)))
                   + _dotg(w1_bot, bu2_ref[...], ((0,), (1,)))
                   + bw1_ref[...])
    m_ref[...] = jnp.full((_B, 1), _NEG, jnp.float32)
    s_ref[...] = jnp.zeros((_B, 1), jnp.float32)

  @pl.when(i < _NB)
  def _scores():
    ids = ids_ref[...]
    ohb = ids == lax.broadcasted_iota(jnp.int32, (_B, 1), 0)
    pre = _dotg(wxt_ref[...], x_ref[...], ((1,), (1,)))
    cg = _dotg(ct_ref[...], ohb.astype(jnp.float32), ((1,), (0,)))
    ht = jnp.tanh(pre + cg)
    et = jnp.sum(ht * ww2_ref[...], axis=0, keepdims=True) + bw2_ref[...]
    e_ref[pl.ds(i, 1), :] = et
    m_part = jnp.max(jnp.where(ohb, et, _NEG), axis=1, keepdims=True)
    s_part = jnp.sum(jnp.where(ohb, jnp.exp(et), 0.0), axis=1, keepdims=True)
    m_ref[...] = jnp.maximum(m_ref[...], m_part)
    s_ref[...] = s_ref[...] + s_part

  @pl.when(i == _NB)
  def _normalize():
    m = jnp.maximum(m_ref[...], -80.0)
    em = jnp.exp(-m)
    s = s_ref[...]
    q = jnp.where(s > 0.0, em / (em * s + 1e-16), 0.0)
    iota_b = lax.broadcasted_iota(jnp.int32, (_B, 1), 0)
    for j in range(_NB):
      ids_j = idsf_ref[0:1, j * _BN:(j + 1) * _BN]
      ohf = (ids_j == iota_b).astype(jnp.float32)
      qg = _dotg(q, ohf, ((0,), (0,)))
      a_ref[0:1, j * _BN:(j + 1) * _BN] = (
          jnp.exp(e_ref[j:j + 1, :]) * qg)


def kernel(x_t, x_t_batch, smile_latent, W_U1, b_U1, W_U2, b_U2,
           W_W1, b_W1, W_W2, b_W2):
  ids = x_t_batch.astype(jnp.int32).reshape(1, _N)
  last = _NB - 1
  alpha = pl.pallas_call(
      _body,
      grid=(_NB + 1,),
      in_specs=[
          pl.BlockSpec((_BN, 93), lambda i: (jnp.minimum(i, last), 0)),
          pl.BlockSpec((1, _BN), lambda i: (0, jnp.minimum(i, last))),
          pl.BlockSpec((1, _N), lambda i: (0, 0)),
          pl.BlockSpec((16, 500), lambda i: (0, 0)),
          pl.BlockSpec((500, 150), lambda i: (0, 0)),
          pl.BlockSpec((1, 150), lambda i: (0, 0)),
          pl.BlockSpec((93, 150), lambda i: (0, 0)),
          pl.BlockSpec((1, 150), lambda i: (0, 0)),
          pl.BlockSpec((300, 150), lambda i: (0, 0)),
          pl.BlockSpec((150, 1), lambda i: (0, 0)),
          pl.BlockSpec((150, 1), lambda i: (0, 0)),
          pl.BlockSpec((1, 1), lambda i: (0, 0)),
      ],
      out_specs=pl.BlockSpec((1, _N), lambda i: (0, 0)),
      out_shape=jax.ShapeDtypeStruct((1, _N), jnp.float32),
      scratch_shapes=[
          pltpu.VMEM((150, 93), jnp.float32),
          pltpu.VMEM((150, _B), jnp.float32),
          pltpu.VMEM((_B, 1), jnp.float32),
          pltpu.VMEM((_B, 1), jnp.float32),
          pltpu.VMEM((_NB, _BN), jnp.float32),
      ],
  )(x_t, ids, ids, smile_latent, W_U1, b_U1.reshape(1, 150), W_U2,
    b_U2.reshape(1, 150), W_W1, b_W1.reshape(150, 1), W_W2,
    b_W2.reshape(1, 1))
  return alpha.reshape(_N, 1)


# speedup vs baseline: 1.0723x; 1.0026x over previous
import jax
import jax.numpy as jnp
from jax import lax
from jax.experimental import pallas as pl
from jax.experimental.pallas import tpu as pltpu

_N = 32768
_B = 16
_BN = 8192
_NB = _N // _BN
_NEG = -1.0e30


def _dotg(a, b, dims):
  return lax.dot_general(a, b, (dims, ((), ())),
                         preferred_element_type=jnp.float32)


def _body(x_ref, ids_ref, idsf_ref, sl_ref, wu1_ref, bu1_ref, wu2_ref,
          bu2_ref, ww1_ref, bw1_ref, ww2_ref, bw2_ref,
          a_ref, wxt_ref, ct_ref, m_ref, s_ref, e_ref):
  i = pl.program_id(0)

  @pl.when(i == 0)
  def _init():
    ww1 = ww1_ref[...]
    w1_top = ww1[:150, :]
    w1_bot = ww1[150:, :]
    wxt_ref[...] = _dotg(w1_bot, wu2_ref[...], ((0,), (1,)))
    u1 = jnp.dot(sl_ref[...], wu1_ref[...],
                 preferred_element_type=jnp.float32) + bu1_ref[...]
    ct_ref[...] = (_dotg(w1_top, u1, ((0,), (1,)))
                   + _dotg(w1_bot, bu2_ref[...], ((0,), (1,)))
                   + bw1_ref[...])
    m_ref[...] = jnp.full((_B, 1), _NEG, jnp.float32)
    s_ref[...] = jnp.zeros((_B, 1), jnp.float32)

  @pl.when(i < _NB)
  def _scores():
    ids = ids_ref[...]
    ohb = ids == lax.broadcasted_iota(jnp.int32, (_B, 1), 0)
    pre = _dotg(wxt_ref[...], x_ref[...], ((1,), (1,)))
    cg = _dotg(ct_ref[...], ohb.astype(jnp.float32), ((1,), (0,)))
    ht = jnp.tanh(pre + cg)
    et = jnp.sum(ht * ww2_ref[...], axis=0, keepdims=True) + bw2_ref[...]
    e_ref[pl.ds(i, 1), :] = et
    m_part = jnp.max(jnp.where(ohb, et, _NEG), axis=1, keepdims=True)
    s_part = jnp.sum(jnp.where(ohb, jnp.exp(et), 0.0), axis=1, keepdims=True)
    m_ref[...] = jnp.maximum(m_ref[...], m_part)
    s_ref[...] = s_ref[...] + s_part

  @pl.when(i == _NB - 1)
  def _normalize():
    m = jnp.maximum(m_ref[...], -80.0)
    em = jnp.exp(-m)
    s = s_ref[...]
    q = jnp.where(s > 0.0, em / (em * s + 1e-16), 0.0)
    iota_b = lax.broadcasted_iota(jnp.int32, (_B, 1), 0)
    for j in range(_NB):
      ids_j = idsf_ref[0:1, j * _BN:(j + 1) * _BN]
      ohf = (ids_j == iota_b).astype(jnp.float32)
      qg = _dotg(q, ohf, ((0,), (0,)))
      a_ref[0:1, j * _BN:(j + 1) * _BN] = (
          jnp.exp(e_ref[j:j + 1, :]) * qg)


def kernel(x_t, x_t_batch, smile_latent, W_U1, b_U1, W_U2, b_U2,
           W_W1, b_W1, W_W2, b_W2):
  ids = x_t_batch.astype(jnp.int32).reshape(1, _N)
  last = _NB - 1
  alpha = pl.pallas_call(
      _body,
      grid=(_NB,),
      in_specs=[
          pl.BlockSpec((_BN, 93), lambda i: (jnp.minimum(i, last), 0)),
          pl.BlockSpec((1, _BN), lambda i: (0, jnp.minimum(i, last))),
          pl.BlockSpec((1, _N), lambda i: (0, 0)),
          pl.BlockSpec((16, 500), lambda i: (0, 0)),
          pl.BlockSpec((500, 150), lambda i: (0, 0)),
          pl.BlockSpec((1, 150), lambda i: (0, 0)),
          pl.BlockSpec((93, 150), lambda i: (0, 0)),
          pl.BlockSpec((1, 150), lambda i: (0, 0)),
          pl.BlockSpec((300, 150), lambda i: (0, 0)),
          pl.BlockSpec((150, 1), lambda i: (0, 0)),
          pl.BlockSpec((150, 1), lambda i: (0, 0)),
          pl.BlockSpec((1, 1), lambda i: (0, 0)),
      ],
      out_specs=pl.BlockSpec((1, _N), lambda i: (0, 0)),
      out_shape=jax.ShapeDtypeStruct((1, _N), jnp.float32),
      scratch_shapes=[
          pltpu.VMEM((150, 93), jnp.float32),
          pltpu.VMEM((150, _B), jnp.float32),
          pltpu.VMEM((_B, 1), jnp.float32),
          pltpu.VMEM((_B, 1), jnp.float32),
          pltpu.VMEM((_NB, _BN), jnp.float32),
      ],
  )(x_t, ids, ids, smile_latent, W_U1, b_U1.reshape(1, 150), W_U2,
    b_U2.reshape(1, 150), W_W1, b_W1.reshape(150, 1), W_W2,
    b_W2.reshape(1, 1))
  return alpha.reshape(_N, 1)
